# baseline (speedup 1.0000x reference)
import functools

import jax
import jax.numpy as jnp
from jax import lax
from jax.experimental import pallas as pl
from jax.experimental.pallas import tpu as pltpu
from jax.experimental.pallas import tpu_sc as plsc

N = 10000
E = 320000
HID = 128
CNT = 16
NUM_LAYERS = 4
NUM_EMB = 121
MAX_NODES = 121
NUM_GRAPHS = 100

F32 = jnp.float32


_NC = 2
_NS = 16
_NT = _NC * _NS
_K = 128
_CPT = 80
_EPT = _CPT * _K
_EPAD = _NT * _EPT
_NPAD = _EPAD - E
_WCH = N // 80
_NPAIR = _CPT // 2


def _make_segsum():
  mesh = plsc.VectorSubcoreMesh(core_axis_name="c", subcore_axis_name="s")

  @functools.partial(
      pl.kernel,
      mesh=mesh,
      out_type=[
          jax.ShapeDtypeStruct((N, HID), F32),
          jax.ShapeDtypeStruct((N, HID), F32),
      ],
      scratch_types=[
          pltpu.VMEM((_CPT, _K), jnp.int32),
          pltpu.VMEM((_K,), jnp.int32),
          pltpu.VMEM((_K,), jnp.int32),
          pltpu.VMEM((_K, HID), F32),
          pltpu.VMEM((_K, HID), F32),
          pltpu.VMEM_SHARED((N, HID), F32),
          pltpu.SemaphoreType.DMA,
          pltpu.SemaphoreType.DMA,
          pltpu.SemaphoreType.DMA,
          pltpu.SemaphoreType.DMA,
      ],
  )
  def segsum(p_hbm, srci_hbm, dsti_hbm, agg0_hbm, agg1_hbm,
             srcall, da, db, ra, rb, acc,
             gsa, gsb, ssa, ssb):
    c = lax.axis_index("c")
    s = lax.axis_index("s")
    t = c * _NS + s

    pltpu.sync_copy(srci_hbm.at[t], srcall)

    def _zrow(r, _):
      def _zcol(j, _):
        ra[r, pl.ds(j * 16, 16)] = jnp.zeros((16,), F32)
        return 0
      lax.fori_loop(0, HID // 16, _zcol, 0)
      return 0
    lax.fori_loop(0, 80, _zrow, 0)
    zsrc = ra.at[pl.ds(0, 80)]

    def _zchunk(j, _):
      idx = s + j * _NS
      @pl.when(idx < _WCH)
      def _():
        pltpu.sync_copy(zsrc, acc.at[pl.ds(idx * 80, 80)])
      return 0
    lax.fori_loop(0, (_WCH + _NS - 1) // _NS, _zchunk, 0)
    plsc.subcore_barrier()

    ebase = t * _EPT

    def _gather(k, buf, dbuf, gsem):
      pltpu.async_copy(p_hbm.at[srcall.at[k]], buf, gsem)
      pltpu.async_copy(dsti_hbm.at[pl.ds(ebase + k * _K, _K)], dbuf, gsem)

    def _gwait(buf, dbuf, gsem):
      pltpu.make_async_copy(p_hbm.at[srcall.at[0]], buf, gsem).wait()
      pltpu.make_async_copy(dsti_hbm.at[pl.ds(0, _K)], dbuf, gsem).wait()

    def _scat(buf, dbuf, ssem):
      return pltpu.async_copy(buf, acc.at[dbuf], ssem, add=True)

    _gather(0, ra, da, gsa)
    _gather(1, rb, db, gsb)

    def _iter(i, _):
      _gwait(ra, da, gsa)
      sa = _scat(ra, da, ssa)
      _gwait(rb, db, gsb)
      sb = _scat(rb, db, ssb)
      sa.wait()
      @pl.when(i < _NPAIR - 1)
      def _():
        _gather(2 * i + 2, ra, da, gsa)
      sb.wait()
      @pl.when(i < _NPAIR - 1)
      def _():
        _gather(2 * i + 3, rb, db, gsb)
      return 0
    lax.fori_loop(0, _NPAIR, _iter, 0)
    plsc.subcore_barrier()

    def _writeout(out_hbm):
      def _w(j, _):
        idx = s + j * _NS
        @pl.when(idx < _WCH)
        def _():
          r0 = idx * 80
          pltpu.sync_copy(acc.at[pl.ds(r0, 80)], zsrc)
          pltpu.sync_copy(zsrc, out_hbm.at[pl.ds(r0, 80)])
        return 0
      lax.fori_loop(0, (_WCH + _NS - 1) // _NS, _w, 0)

    @pl.when(c == 0)
    def _():
      _writeout(agg0_hbm)

    @pl.when(c == 1)
    def _():
      _writeout(agg1_hbm)

  return segsum


_segsum_call = None


def _segsum(p, srci, dsti):
  global _segsum_call
  if _segsum_call is None:
    _segsum_call = _make_segsum()
  return _segsum_call(p, srci, dsti)


def _enc_body(x_ref, counts_ref, uc_ref, emb_ref, cW1_ref, cb1_ref,
              cW2_ref, cb2_ref, W1a_ref, W1b_ref, out_ref):
  xi = x_ref[...]
  onehot = (lax.broadcasted_iota(jnp.int32, (N, NUM_EMB), 1) == xi
            ).astype(F32)
  he = jnp.dot(onehot, emb_ref[...], preferred_element_type=F32)
  ch = jnp.maximum(
      jnp.dot(counts_ref[...], cW1_ref[...], preferred_element_type=F32)
      + cb1_ref[...], 0.0)
  ch = jnp.dot(ch, cW2_ref[...], preferred_element_type=F32) + cb2_ref[...]
  ch = ch * uc_ref[0, 0]
  out_ref[...] = (
      jnp.dot(he, W1a_ref[...], preferred_element_type=F32)
      + jnp.dot(ch, W1b_ref[...], preferred_element_type=F32))


def _layer_body(*refs, residual, last):
  if residual:
    h_ref, p_ref, a0_ref, a1_ref, b1_ref, W2_ref, b2_ref, g_ref, be_ref = \
        refs[:9]
    rest = refs[9:]
  else:
    p_ref, a0_ref, a1_ref, b1_ref, W2_ref, b2_ref, g_ref, be_ref = refs[:8]
    rest = refs[8:]
  if last:
    (out_ref,) = rest
  else:
    W1n_ref, out_ref, pout_ref = rest

  p = p_ref[...]
  corr = (lax.broadcasted_iota(jnp.int32, (N, 1), 0) < _NPAD).astype(F32)
  a1 = a1_ref[...] - corr * p[0:1, :]
  z = jnp.maximum(p + a0_ref[...] + a1 + b1_ref[...], 0.0)
  z = jnp.dot(z, W2_ref[...], preferred_element_type=F32) + b2_ref[...]
  mu = jnp.mean(z, axis=0, keepdims=True)
  var = jnp.mean(jnp.square(z - mu), axis=0, keepdims=True)
  z = g_ref[...] * (z - mu) * lax.rsqrt(var + 1e-5) + be_ref[...]
  z = jnp.maximum(z, 0.0)
  if residual:
    z = z + h_ref[...]
  out_ref[...] = z
  if not last:
    pout_ref[...] = jnp.dot(z, W1n_ref[...], preferred_element_type=F32)


def _final_body(h_ref, batch_ref, dW1_ref, db1_ref, dW2_ref, db2_ref,
                out_ref):
  h = h_ref[...]
  z = jnp.maximum(
      jnp.dot(h, dW1_ref[...], preferred_element_type=F32) + db1_ref[...],
      0.0)
  z = jnp.dot(z, dW2_ref[...], preferred_element_type=F32) + db2_ref[...]
  preds = 1.0 / (1.0 + jnp.exp(-z))

  b = batch_ref[...]
  Bh = (lax.broadcasted_iota(jnp.int32, (N, NUM_GRAPHS), 1) == b
        ).astype(F32)
  cnts = jnp.sum(Bh, axis=0, keepdims=True)
  tri = (lax.broadcasted_iota(jnp.int32, (NUM_GRAPHS, NUM_GRAPHS), 0)
         < lax.broadcasted_iota(jnp.int32, (NUM_GRAPHS, NUM_GRAPHS), 1)
         ).astype(F32)
  offs = jnp.dot(cnts, tri, preferred_element_type=F32)
  off_node = lax.dot_general(Bh, offs, (((1,), (1,)), ((), ())),
                             preferred_element_type=F32)
  rowid = lax.broadcasted_iota(jnp.int32, (N, 1), 0).astype(F32)
  pos = rowid - off_node
  mask = pos < float(MAX_NODES)
  Pm = ((lax.broadcasted_iota(jnp.int32, (N, MAX_NODES), 1).astype(F32)
         == pos) & mask).astype(F32)
  out = lax.dot_general(Bh, Pm * preds, (((0,), (0,)), ((), ())),
                        preferred_element_type=F32)
  out_ref[...] = out


def _tc_call(body, out_shape):
  return pl.pallas_call(body, out_shape=out_shape)


def kernel(x, edge_index, counts, use_counts, batch, emb, cW1, cb1, cW2, cb2,
           conv_W1, conv_b1, conv_W2, conv_b2, conv_gamma, conv_beta,
           dW1, db1, dW2, db2):
  x2 = x.reshape(N, 1)
  batch2 = batch.reshape(N, 1)
  uc = jnp.asarray(use_counts, F32).reshape(1, 1)
  srci = jnp.concatenate(
      [edge_index[0], jnp.zeros((_NPAD,), jnp.int32)]).reshape(_NT, _CPT, _K)
  dsti = jnp.concatenate(
      [edge_index[1], jnp.arange(_NPAD, dtype=jnp.int32)])
  W1a = conv_W1[0][:HID]
  W1b = conv_W1[0][HID:]

  p = _tc_call(_enc_body, jax.ShapeDtypeStruct((N, HID), F32))(
      x2, counts, uc, emb, cW1, cb1.reshape(1, -1), cW2, cb2.reshape(1, -1),
      W1a, W1b)

  h = None
  for i in range(NUM_LAYERS):
    agg0, agg1 = _segsum(p, srci, dsti)
    residual = i > 0
    last = i == NUM_LAYERS - 1
    body = functools.partial(_layer_body, residual=residual, last=last)
    if last:
      out_shape = jax.ShapeDtypeStruct((N, HID), F32)
    else:
      out_shape = (jax.ShapeDtypeStruct((N, HID), F32),
                   jax.ShapeDtypeStruct((N, HID), F32))
    args = []
    if residual:
      args.append(h)
    args += [p, agg0, agg1, conv_b1[i].reshape(1, -1), conv_W2[i],
             conv_b2[i].reshape(1, -1), conv_gamma[i].reshape(1, -1),
             conv_beta[i].reshape(1, -1)]
    if not last:
      args.append(conv_W1[i + 1])
      h, p = _tc_call(body, out_shape)(*args)
    else:
      h = _tc_call(body, out_shape)(*args)

  out = _tc_call(_final_body,
                 jax.ShapeDtypeStruct((NUM_GRAPHS, MAX_NODES), F32))(
      h, batch2, dW1, db1.reshape(1, -1), dW2, db2.reshape(1, -1))
  return out


# speedup vs baseline: 3.3090x; 3.3090x over previous
import functools

import jax
import jax.numpy as jnp
from jax import lax
from jax.experimental import pallas as pl
from jax.experimental.pallas import tpu as pltpu
from jax.experimental.pallas import tpu_sc as plsc

N = 10000
E = 320000
HID = 128
CNT = 16
NUM_LAYERS = 4
NUM_EMB = 121
MAX_NODES = 121
NUM_GRAPHS = 100

F32 = jnp.float32


_NC = 2
_NS = 16
_NT = _NC * _NS
_K = 128
_CPT = 80
_EPT = _CPT * _K
_EPAD = _NT * _EPT
_NPAD = _EPAD - E
_WCH = N // 80
_NPAIR = _CPT // 2


def _make_segsum():
  mesh = plsc.VectorSubcoreMesh(core_axis_name="c", subcore_axis_name="s")

  @functools.partial(
      pl.kernel,
      mesh=mesh,
      out_type=[
          jax.ShapeDtypeStruct((N, HID), F32),
          jax.ShapeDtypeStruct((N, HID), F32),
      ],
      scratch_types=[
          pltpu.VMEM((_CPT, _K), jnp.int32),
          pltpu.VMEM((_K,), jnp.int32),
          pltpu.VMEM((_K,), jnp.int32),
          pltpu.VMEM((_K, HID), F32),
          pltpu.VMEM((_K, HID), F32),
          pltpu.VMEM_SHARED((N, HID), F32),
          pltpu.SemaphoreType.DMA,
          pltpu.SemaphoreType.DMA,
          pltpu.SemaphoreType.DMA,
          pltpu.SemaphoreType.DMA,
      ],
  )
  def segsum(p_hbm, srci_hbm, dsti_hbm, agg0_hbm, agg1_hbm,
             srcall, da, db, ra, rb, acc,
             gsa, gsb, ssa, ssb):
    c = lax.axis_index("c")
    s = lax.axis_index("s")
    t = c * _NS + s

    pltpu.sync_copy(srci_hbm.at[t], srcall)

    def _zrow(r, _):
      def _zcol(j, _):
        ra[r, pl.ds(j * 16, 16)] = jnp.zeros((16,), F32)
        return 0
      lax.fori_loop(0, HID // 16, _zcol, 0)
      return 0
    lax.fori_loop(0, 80, _zrow, 0)
    zsrc = ra.at[pl.ds(0, 80)]

    def _zchunk(j, _):
      idx = s + j * _NS
      @pl.when(idx < _WCH)
      def _():
        pltpu.sync_copy(zsrc, acc.at[pl.ds(idx * 80, 80)])
      return 0
    lax.fori_loop(0, (_WCH + _NS - 1) // _NS, _zchunk, 0)
    plsc.subcore_barrier()

    ebase = t * _EPT

    def _gather(k, buf, dbuf, gsem):
      pltpu.async_copy(p_hbm.at[srcall.at[k]], buf, gsem)
      pltpu.async_copy(dsti_hbm.at[pl.ds(ebase + k * _K, _K)], dbuf, gsem)

    def _gwait(buf, dbuf, gsem):
      pltpu.make_async_copy(p_hbm.at[srcall.at[0]], buf, gsem).wait()
      pltpu.make_async_copy(dsti_hbm.at[pl.ds(0, _K)], dbuf, gsem).wait()

    def _scat(buf, dbuf, ssem):
      return pltpu.async_copy(buf, acc.at[dbuf], ssem, add=True)

    _gather(0, ra, da, gsa)
    _gather(1, rb, db, gsb)

    def _iter(i, _):
      _gwait(ra, da, gsa)
      sa = _scat(ra, da, ssa)
      _gwait(rb, db, gsb)
      sb = _scat(rb, db, ssb)
      sa.wait()
      @pl.when(i < _NPAIR - 1)
      def _():
        _gather(2 * i + 2, ra, da, gsa)
      sb.wait()
      @pl.when(i < _NPAIR - 1)
      def _():
        _gather(2 * i + 3, rb, db, gsb)
      return 0
    lax.fori_loop(0, _NPAIR, _iter, 0)
    plsc.subcore_barrier()

    def _writeout(out_hbm):
      def _w(j, _):
        idx = s + j * _NS
        @pl.when(idx < _WCH)
        def _():
          r0 = idx * 80
          pltpu.sync_copy(acc.at[pl.ds(r0, 80)], zsrc)
          pltpu.sync_copy(zsrc, out_hbm.at[pl.ds(r0, 80)])
        return 0
      lax.fori_loop(0, (_WCH + _NS - 1) // _NS, _w, 0)

    @pl.when(c == 0)
    def _():
      _writeout(agg0_hbm)

    @pl.when(c == 1)
    def _():
      _writeout(agg1_hbm)

  return segsum


_segsum_call = None


def _segsum(p, srci, dsti):
  global _segsum_call
  if _segsum_call is None:
    _segsum_call = _make_segsum()
  return _segsum_call(p, srci, dsti)


def _enc_body(x_ref, counts_ref, uc_ref, emb_ref, cW1_ref, cb1_ref,
              cW2_ref, cb2_ref, W1a_ref, W1b_ref, out_ref):
  xi = x_ref[...]
  onehot = (lax.broadcasted_iota(jnp.int32, (N, NUM_EMB), 1) == xi
            ).astype(F32)
  he = jnp.dot(onehot, emb_ref[...], preferred_element_type=F32)
  ch = jnp.maximum(
      jnp.dot(counts_ref[...], cW1_ref[...], preferred_element_type=F32)
      + cb1_ref[...], 0.0)
  ch = jnp.dot(ch, cW2_ref[...], preferred_element_type=F32) + cb2_ref[...]
  ch = ch * uc_ref[0, 0]
  out_ref[...] = (
      jnp.dot(he, W1a_ref[...], preferred_element_type=F32)
      + jnp.dot(ch, W1b_ref[...], preferred_element_type=F32))


def _layer_body(*refs, residual, last):
  if residual:
    h_ref, p_ref, a0_ref, a1_ref, b1_ref, W2_ref, b2_ref, g_ref, be_ref = \
        refs[:9]
    rest = refs[9:]
  else:
    p_ref, a0_ref, a1_ref, b1_ref, W2_ref, b2_ref, g_ref, be_ref = refs[:8]
    rest = refs[8:]
  if last:
    (out_ref,) = rest
  else:
    W1n_ref, out_ref, pout_ref = rest

  p = p_ref[...]
  corr = (lax.broadcasted_iota(jnp.int32, (N, 1), 0) < _NPAD).astype(F32)
  a1 = a1_ref[...] - corr * p
  z = jnp.maximum(p + a0_ref[...] + a1 + b1_ref[...], 0.0)
  z = jnp.dot(z, W2_ref[...], preferred_element_type=F32) + b2_ref[...]
  mu = jnp.mean(z, axis=0, keepdims=True)
  var = jnp.mean(jnp.square(z - mu), axis=0, keepdims=True)
  z = g_ref[...] * (z - mu) * lax.rsqrt(var + 1e-5) + be_ref[...]
  z = jnp.maximum(z, 0.0)
  if residual:
    z = z + h_ref[...]
  out_ref[...] = z
  if not last:
    pout_ref[...] = jnp.dot(z, W1n_ref[...], preferred_element_type=F32)


def _final_body(h_ref, batch_ref, dW1_ref, db1_ref, dW2_ref, db2_ref,
                out_ref):
  h = h_ref[...]
  z = jnp.maximum(
      jnp.dot(h, dW1_ref[...], preferred_element_type=F32) + db1_ref[...],
      0.0)
  z = jnp.dot(z, dW2_ref[...], preferred_element_type=F32) + db2_ref[...]
  preds = 1.0 / (1.0 + jnp.exp(-z))

  b = batch_ref[...]
  Bh = (lax.broadcasted_iota(jnp.int32, (N, NUM_GRAPHS), 1) == b
        ).astype(F32)
  cnts = jnp.sum(Bh, axis=0, keepdims=True)
  tri = (lax.broadcasted_iota(jnp.int32, (NUM_GRAPHS, NUM_GRAPHS), 0)
         < lax.broadcasted_iota(jnp.int32, (NUM_GRAPHS, NUM_GRAPHS), 1)
         ).astype(F32)
  offs = jnp.dot(cnts, tri, preferred_element_type=F32)
  off_node = lax.dot_general(Bh, offs, (((1,), (1,)), ((), ())),
                             preferred_element_type=F32)
  rowid = lax.broadcasted_iota(jnp.int32, (N, 1), 0).astype(F32)
  pos = rowid - off_node
  mask = pos < float(MAX_NODES)
  Pm = ((lax.broadcasted_iota(jnp.int32, (N, MAX_NODES), 1).astype(F32)
         == pos) & mask).astype(F32)
  out = lax.dot_general(Bh, Pm * preds, (((0,), (0,)), ((), ())),
                        preferred_element_type=F32)
  out_ref[...] = out


def _tc_call(body, out_shape):
  return pl.pallas_call(body, out_shape=out_shape)


def kernel(x, edge_index, counts, use_counts, batch, emb, cW1, cb1, cW2, cb2,
           conv_W1, conv_b1, conv_W2, conv_b2, conv_gamma, conv_beta,
           dW1, db1, dW2, db2):
  x2 = x.reshape(N, 1)
  batch2 = batch.reshape(N, 1)
  uc = jnp.asarray(use_counts, F32).reshape(1, 1)
  pad = jnp.arange(_NPAD, dtype=jnp.int32)
  srci = jnp.concatenate([edge_index[0], pad]).reshape(_NT, _CPT, _K)
  dsti = jnp.concatenate([edge_index[1], pad])
  W1a = conv_W1[0][:HID]
  W1b = conv_W1[0][HID:]

  p = _tc_call(_enc_body, jax.ShapeDtypeStruct((N, HID), F32))(
      x2, counts, uc, emb, cW1, cb1.reshape(1, -1), cW2, cb2.reshape(1, -1),
      W1a, W1b)

  h = None
  for i in range(NUM_LAYERS):
    agg0, agg1 = _segsum(p, srci, dsti)
    residual = i > 0
    last = i == NUM_LAYERS - 1
    body = functools.partial(_layer_body, residual=residual, last=last)
    if last:
      out_shape = jax.ShapeDtypeStruct((N, HID), F32)
    else:
      out_shape = (jax.ShapeDtypeStruct((N, HID), F32),
                   jax.ShapeDtypeStruct((N, HID), F32))
    args = []
    if residual:
      args.append(h)
    args += [p, agg0, agg1, conv_b1[i].reshape(1, -1), conv_W2[i],
             conv_b2[i].reshape(1, -1), conv_gamma[i].reshape(1, -1),
             conv_beta[i].reshape(1, -1)]
    if not last:
      args.append(conv_W1[i + 1])
      h, p = _tc_call(body, out_shape)(*args)
    else:
      h = _tc_call(body, out_shape)(*args)

  out = _tc_call(_final_body,
                 jax.ShapeDtypeStruct((NUM_GRAPHS, MAX_NODES), F32))(
      h, batch2, dW1, db1.reshape(1, -1), dW2, db2.reshape(1, -1))
  return out


# speedup vs baseline: 3.6969x; 1.1172x over previous
import functools

import jax
import jax.numpy as jnp
from jax import lax
from jax.experimental import pallas as pl
from jax.experimental.pallas import tpu as pltpu
from jax.experimental.pallas import tpu_sc as plsc

N = 10000
E = 320000
HID = 128
CNT = 16
NUM_LAYERS = 4
NUM_EMB = 121
MAX_NODES = 121
NUM_GRAPHS = 100

F32 = jnp.float32


_NC = 2
_NS = 16
_NT = _NC * _NS
_K = 64
_NSLOT = 3
_CPT = 159
_EPT = _CPT * _K
_EPAD = _NT * _EPT
_NPAD = _EPAD - E
_WCH = N // 80
_NITER = _CPT // _NSLOT


def _make_segsum():
  mesh = plsc.VectorSubcoreMesh(core_axis_name="c", subcore_axis_name="s")

  @functools.partial(
      pl.kernel,
      mesh=mesh,
      out_type=[
          jax.ShapeDtypeStruct((N, HID), F32),
          jax.ShapeDtypeStruct((N, HID), F32),
      ],
      scratch_types=(
          [pltpu.VMEM((_CPT, _K), jnp.int32)]
          + [pltpu.VMEM((_K,), jnp.int32)] * _NSLOT
          + [pltpu.VMEM((_K, HID), F32)] * _NSLOT
          + [pltpu.VMEM_SHARED((N, HID), F32)]
          + [pltpu.SemaphoreType.DMA] * (2 * _NSLOT)
      ),
  )
  def segsum(p_hbm, srci_hbm, dsti_hbm, agg0_hbm, agg1_hbm,
             srcall, *rest):
    dbufs = rest[:_NSLOT]
    rbufs = rest[_NSLOT:2 * _NSLOT]
    acc = rest[2 * _NSLOT]
    gsems = rest[2 * _NSLOT + 1:3 * _NSLOT + 1]
    ssems = rest[3 * _NSLOT + 1:4 * _NSLOT + 1]
    c = lax.axis_index("c")
    s = lax.axis_index("s")
    t = c * _NS + s

    pltpu.sync_copy(srci_hbm.at[t], srcall)

    def _zrow(r, _):
      def _zcol(j, _):
        rbufs[0][r, pl.ds(j * 16, 16)] = jnp.zeros((16,), F32)
        return 0
      lax.fori_loop(0, HID // 16, _zcol, 0)
      return 0
    lax.fori_loop(0, 40, _zrow, 0)
    zsrc = rbufs[0].at[pl.ds(0, 40)]
    _ZN = N // 40

    def _zchunk(j, _):
      idx = s + j * _NS
      @pl.when(idx < _ZN)
      def _():
        pltpu.sync_copy(zsrc, acc.at[pl.ds(idx * 40, 40)])
      return 0
    lax.fori_loop(0, (_ZN + _NS - 1) // _NS, _zchunk, 0)
    plsc.subcore_barrier()

    ebase = t * _EPT

    def _gather(k, j):
      pltpu.async_copy(p_hbm.at[srcall.at[k]], rbufs[j], gsems[j])
      pltpu.async_copy(dsti_hbm.at[pl.ds(ebase + k * _K, _K)], dbufs[j],
                       gsems[j])

    def _gwait(j):
      pltpu.make_async_copy(p_hbm.at[srcall.at[0]], rbufs[j],
                            gsems[j]).wait()
      pltpu.make_async_copy(dsti_hbm.at[pl.ds(0, _K)], dbufs[j],
                            gsems[j]).wait()

    def _scat(j):
      return pltpu.async_copy(rbufs[j], acc.at[dbufs[j]], ssems[j],
                              add=True)

    for j in range(_NSLOT):
      _gather(j, j)

    def _iter(i, _):
      k = _NSLOT * i
      handles = []
      for j in range(_NSLOT):
        _gwait(j)
        handles.append(_scat(j))
      for j in range(_NSLOT):
        handles[j].wait()
        @pl.when(i < _NITER - 1)
        def _(j=j):
          _gather(k + _NSLOT + j, j)
      return 0
    lax.fori_loop(0, _NITER, _iter, 0)
    plsc.subcore_barrier()

    def _writeout(out_hbm):
      def _w(j, _):
        idx = s + j * _NS
        @pl.when(idx < _ZN)
        def _():
          r0 = idx * 40
          pltpu.sync_copy(acc.at[pl.ds(r0, 40)], zsrc)
          pltpu.sync_copy(zsrc, out_hbm.at[pl.ds(r0, 40)])
        return 0
      lax.fori_loop(0, (_ZN + _NS - 1) // _NS, _w, 0)

    @pl.when(c == 0)
    def _():
      _writeout(agg0_hbm)

    @pl.when(c == 1)
    def _():
      _writeout(agg1_hbm)

  return segsum


_segsum_call = None


def _segsum(p, srci, dsti):
  global _segsum_call
  if _segsum_call is None:
    _segsum_call = _make_segsum()
  return _segsum_call(p, srci, dsti)


def _enc_body(x_ref, counts_ref, uc_ref, emb_ref, cW1_ref, cb1_ref,
              cW2_ref, cb2_ref, W1a_ref, W1b_ref, out_ref):
  xi = x_ref[...]
  onehot = (lax.broadcasted_iota(jnp.int32, (N, NUM_EMB), 1) == xi
            ).astype(F32)
  he = jnp.dot(onehot, emb_ref[...], preferred_element_type=F32)
  ch = jnp.maximum(
      jnp.dot(counts_ref[...], cW1_ref[...], preferred_element_type=F32)
      + cb1_ref[...], 0.0)
  ch = jnp.dot(ch, cW2_ref[...], preferred_element_type=F32) + cb2_ref[...]
  ch = ch * uc_ref[0, 0]
  out_ref[...] = (
      jnp.dot(he, W1a_ref[...], preferred_element_type=F32)
      + jnp.dot(ch, W1b_ref[...], preferred_element_type=F32))


def _layer_body(*refs, residual, last):
  if residual:
    h_ref, p_ref, a0_ref, a1_ref, b1_ref, W2_ref, b2_ref, g_ref, be_ref = \
        refs[:9]
    rest = refs[9:]
  else:
    p_ref, a0_ref, a1_ref, b1_ref, W2_ref, b2_ref, g_ref, be_ref = refs[:8]
    rest = refs[8:]
  if last:
    (out_ref,) = rest
  else:
    W1n_ref, out_ref, pout_ref = rest

  p = p_ref[...]
  corr = (lax.broadcasted_iota(jnp.int32, (N, 1), 0) < _NPAD).astype(F32)
  a1 = a1_ref[...] - corr * p
  z = jnp.maximum(p + a0_ref[...] + a1 + b1_ref[...], 0.0)
  z = jnp.dot(z, W2_ref[...], preferred_element_type=F32) + b2_ref[...]
  mu = jnp.mean(z, axis=0, keepdims=True)
  var = jnp.mean(jnp.square(z - mu), axis=0, keepdims=True)
  z = g_ref[...] * (z - mu) * lax.rsqrt(var + 1e-5) + be_ref[...]
  z = jnp.maximum(z, 0.0)
  if residual:
    z = z + h_ref[...]
  out_ref[...] = z
  if not last:
    pout_ref[...] = jnp.dot(z, W1n_ref[...], preferred_element_type=F32)


def _final_body(h_ref, batch_ref, dW1_ref, db1_ref, dW2_ref, db2_ref,
                out_ref):
  h = h_ref[...]
  z = jnp.maximum(
      jnp.dot(h, dW1_ref[...], preferred_element_type=F32) + db1_ref[...],
      0.0)
  z = jnp.dot(z, dW2_ref[...], preferred_element_type=F32) + db2_ref[...]
  preds = 1.0 / (1.0 + jnp.exp(-z))

  b = batch_ref[...]
  Bh = (lax.broadcasted_iota(jnp.int32, (N, NUM_GRAPHS), 1) == b
        ).astype(F32)
  cnts = jnp.sum(Bh, axis=0, keepdims=True)
  tri = (lax.broadcasted_iota(jnp.int32, (NUM_GRAPHS, NUM_GRAPHS), 0)
         < lax.broadcasted_iota(jnp.int32, (NUM_GRAPHS, NUM_GRAPHS), 1)
         ).astype(F32)
  offs = jnp.dot(cnts, tri, preferred_element_type=F32)
  off_node = lax.dot_general(Bh, offs, (((1,), (1,)), ((), ())),
                             preferred_element_type=F32)
  rowid = lax.broadcasted_iota(jnp.int32, (N, 1), 0).astype(F32)
  pos = rowid - off_node
  mask = pos < float(MAX_NODES)
  Pm = ((lax.broadcasted_iota(jnp.int32, (N, MAX_NODES), 1).astype(F32)
         == pos) & mask).astype(F32)
  out = lax.dot_general(Bh, Pm * preds, (((0,), (0,)), ((), ())),
                        preferred_element_type=F32)
  out_ref[...] = out


def _tc_call(body, out_shape):
  return pl.pallas_call(body, out_shape=out_shape)


def kernel(x, edge_index, counts, use_counts, batch, emb, cW1, cb1, cW2, cb2,
           conv_W1, conv_b1, conv_W2, conv_b2, conv_gamma, conv_beta,
           dW1, db1, dW2, db2):
  x2 = x.reshape(N, 1)
  batch2 = batch.reshape(N, 1)
  uc = jnp.asarray(use_counts, F32).reshape(1, 1)
  pad = jnp.arange(_NPAD, dtype=jnp.int32)
  srci = jnp.concatenate([edge_index[0], pad]).reshape(_NT, _CPT, _K)
  dsti = jnp.concatenate([edge_index[1], pad])
  W1a = conv_W1[0][:HID]
  W1b = conv_W1[0][HID:]

  p = _tc_call(_enc_body, jax.ShapeDtypeStruct((N, HID), F32))(
      x2, counts, uc, emb, cW1, cb1.reshape(1, -1), cW2, cb2.reshape(1, -1),
      W1a, W1b)

  h = None
  for i in range(NUM_LAYERS):
    agg0, agg1 = _segsum(p, srci, dsti)
    residual = i > 0
    last = i == NUM_LAYERS - 1
    body = functools.partial(_layer_body, residual=residual, last=last)
    if last:
      out_shape = jax.ShapeDtypeStruct((N, HID), F32)
    else:
      out_shape = (jax.ShapeDtypeStruct((N, HID), F32),
                   jax.ShapeDtypeStruct((N, HID), F32))
    args = []
    if residual:
      args.append(h)
    args += [p, agg0, agg1, conv_b1[i].reshape(1, -1), conv_W2[i],
             conv_b2[i].reshape(1, -1), conv_gamma[i].reshape(1, -1),
             conv_beta[i].reshape(1, -1)]
    if not last:
      args.append(conv_W1[i + 1])
      h, p = _tc_call(body, out_shape)(*args)
    else:
      h = _tc_call(body, out_shape)(*args)

  out = _tc_call(_final_body,
                 jax.ShapeDtypeStruct((NUM_GRAPHS, MAX_NODES), F32))(
      h, batch2, dW1, db1.reshape(1, -1), dW2, db2.reshape(1, -1))
  return out


# speedup vs baseline: 3.7894x; 1.0250x over previous
import functools

import jax
import jax.numpy as jnp
from jax import lax
from jax.experimental import pallas as pl
from jax.experimental.pallas import tpu as pltpu
from jax.experimental.pallas import tpu_sc as plsc

N = 10000
E = 320000
HID = 128
CNT = 16
NUM_LAYERS = 4
NUM_EMB = 121
MAX_NODES = 121
NUM_GRAPHS = 100

F32 = jnp.float32


_NC = 2
_NS = 16
_NT = _NC * _NS
_K = 64
_NSLOT = 3
_CPT = 159
_EPT = _CPT * _K
_EPAD = _NT * _EPT
_NPAD = _EPAD - E
_WCH = N // 80
_NITER = _CPT // _NSLOT


def _make_segsum():
  mesh = plsc.VectorSubcoreMesh(core_axis_name="c", subcore_axis_name="s")

  @functools.partial(
      pl.kernel,
      mesh=mesh,
      out_type=[
          jax.ShapeDtypeStruct((N, HID), F32),
          jax.ShapeDtypeStruct((N, HID), F32),
      ],
      scratch_types=(
          [pltpu.VMEM((_CPT, _K), jnp.int32)]
          + [pltpu.VMEM((_K,), jnp.int32)] * _NSLOT
          + [pltpu.VMEM((_K, HID), F32)] * _NSLOT
          + [pltpu.VMEM((40, HID), F32)]
          + [pltpu.VMEM_SHARED((N, HID), F32)]
          + [pltpu.SemaphoreType.DMA] * (2 * _NSLOT)
          + [pltpu.SemaphoreType.DMA]
      ),
  )
  def segsum(p_hbm, srci_hbm, dsti_hbm, agg0_hbm, agg1_hbm,
             srcall, *rest):
    dbufs = rest[:_NSLOT]
    rbufs = rest[_NSLOT:2 * _NSLOT]
    zbuf = rest[2 * _NSLOT]
    acc = rest[2 * _NSLOT + 1]
    gsems = rest[2 * _NSLOT + 2:3 * _NSLOT + 2]
    ssems = rest[3 * _NSLOT + 2:4 * _NSLOT + 2]
    zsem = rest[4 * _NSLOT + 2]
    c = lax.axis_index("c")
    s = lax.axis_index("s")
    t = c * _NS + s

    pltpu.sync_copy(srci_hbm.at[t], srcall)

    ebase = t * _EPT

    def _gather(k, j):
      pltpu.async_copy(p_hbm.at[srcall.at[k]], rbufs[j], gsems[j])
      pltpu.async_copy(dsti_hbm.at[pl.ds(ebase + k * _K, _K)], dbufs[j],
                       gsems[j])

    def _gwait(j):
      pltpu.make_async_copy(p_hbm.at[srcall.at[0]], rbufs[j],
                            gsems[j]).wait()
      pltpu.make_async_copy(dsti_hbm.at[pl.ds(0, _K)], dbufs[j],
                            gsems[j]).wait()

    def _scat(j):
      return pltpu.async_copy(rbufs[j], acc.at[dbufs[j]], ssems[j],
                              add=True)

    for j in range(_NSLOT):
      _gather(j, j)

    def _zrow(r, _):
      def _zcol(j, _):
        zbuf[r, pl.ds(j * 16, 16)] = jnp.zeros((16,), F32)
        return 0
      lax.fori_loop(0, HID // 16, _zcol, 0)
      return 0
    lax.fori_loop(0, 40, _zrow, 0)
    _ZN = N // 40
    _ZIT = (_ZN + _NS - 1) // _NS

    def _zchunk(j, _):
      idx = s + j * _NS
      @pl.when(idx < _ZN)
      def _():
        pltpu.async_copy(zbuf, acc.at[pl.ds(idx * 40, 40)], zsem)
      return 0
    lax.fori_loop(0, _ZIT, _zchunk, 0)

    def _zdrain(j, _):
      idx = s + j * _NS
      @pl.when(idx < _ZN)
      def _():
        pltpu.make_async_copy(zbuf, acc.at[pl.ds(0, 40)], zsem).wait()
      return 0
    lax.fori_loop(0, _ZIT, _zdrain, 0)
    plsc.subcore_barrier()

    def _iter(i, _):
      k = _NSLOT * i
      handles = []
      for j in range(_NSLOT):
        _gwait(j)
        handles.append(_scat(j))
      for j in range(_NSLOT):
        handles[j].wait()
        @pl.when(i < _NITER - 1)
        def _(j=j):
          _gather(k + _NSLOT + j, j)
      return 0
    lax.fori_loop(0, _NITER, _iter, 0)
    plsc.subcore_barrier()

    def _writeout(out_hbm):
      def _w(j, _):
        idx = s + j * _NS
        @pl.when(idx < _ZN)
        def _():
          r0 = idx * 40
          pltpu.async_copy(acc.at[pl.ds(r0, 40)], out_hbm.at[pl.ds(r0, 40)],
                           zsem)
        return 0
      lax.fori_loop(0, _ZIT, _w, 0)

      def _wd(j, _):
        idx = s + j * _NS
        @pl.when(idx < _ZN)
        def _():
          pltpu.make_async_copy(acc.at[pl.ds(0, 40)],
                                out_hbm.at[pl.ds(0, 40)], zsem).wait()
        return 0
      lax.fori_loop(0, _ZIT, _wd, 0)

    @pl.when(c == 0)
    def _():
      _writeout(agg0_hbm)

    @pl.when(c == 1)
    def _():
      _writeout(agg1_hbm)

  return segsum


_segsum_call = None


def _segsum(p, srci, dsti):
  global _segsum_call
  if _segsum_call is None:
    _segsum_call = _make_segsum()
  return _segsum_call(p, srci, dsti)


def _enc_body(x_ref, counts_ref, uc_ref, emb_ref, cW1_ref, cb1_ref,
              cW2_ref, cb2_ref, W1a_ref, W1b_ref, out_ref):
  xi = x_ref[...]
  onehot = (lax.broadcasted_iota(jnp.int32, (N, NUM_EMB), 1) == xi
            ).astype(F32)
  he = jnp.dot(onehot, emb_ref[...], preferred_element_type=F32)
  ch = jnp.maximum(
      jnp.dot(counts_ref[...], cW1_ref[...], preferred_element_type=F32)
      + cb1_ref[...], 0.0)
  ch = jnp.dot(ch, cW2_ref[...], preferred_element_type=F32) + cb2_ref[...]
  ch = ch * uc_ref[0, 0]
  out_ref[...] = (
      jnp.dot(he, W1a_ref[...], preferred_element_type=F32)
      + jnp.dot(ch, W1b_ref[...], preferred_element_type=F32))


def _layer_body(*refs, residual, last):
  if residual:
    h_ref, p_ref, a0_ref, a1_ref, b1_ref, W2_ref, b2_ref, g_ref, be_ref = \
        refs[:9]
    rest = refs[9:]
  else:
    p_ref, a0_ref, a1_ref, b1_ref, W2_ref, b2_ref, g_ref, be_ref = refs[:8]
    rest = refs[8:]
  if last:
    (out_ref,) = rest
  else:
    W1n_ref, out_ref, pout_ref = rest

  p = p_ref[...]
  corr = (lax.broadcasted_iota(jnp.int32, (N, 1), 0) < _NPAD).astype(F32)
  a1 = a1_ref[...] - corr * p
  z = jnp.maximum(p + a0_ref[...] + a1 + b1_ref[...], 0.0)
  z = jnp.dot(z, W2_ref[...], preferred_element_type=F32) + b2_ref[...]
  mu = jnp.mean(z, axis=0, keepdims=True)
  var = jnp.mean(jnp.square(z - mu), axis=0, keepdims=True)
  z = g_ref[...] * (z - mu) * lax.rsqrt(var + 1e-5) + be_ref[...]
  z = jnp.maximum(z, 0.0)
  if residual:
    z = z + h_ref[...]
  out_ref[...] = z
  if not last:
    pout_ref[...] = jnp.dot(z, W1n_ref[...], preferred_element_type=F32)


def _final_body(h_ref, batch_ref, dW1_ref, db1_ref, dW2_ref, db2_ref,
                out_ref):
  h = h_ref[...]
  z = jnp.maximum(
      jnp.dot(h, dW1_ref[...], preferred_element_type=F32) + db1_ref[...],
      0.0)
  z = jnp.dot(z, dW2_ref[...], preferred_element_type=F32) + db2_ref[...]
  preds = 1.0 / (1.0 + jnp.exp(-z))

  b = batch_ref[...]
  Bh = (lax.broadcasted_iota(jnp.int32, (N, NUM_GRAPHS), 1) == b
        ).astype(F32)
  cnts = jnp.sum(Bh, axis=0, keepdims=True)
  tri = (lax.broadcasted_iota(jnp.int32, (NUM_GRAPHS, NUM_GRAPHS), 0)
         < lax.broadcasted_iota(jnp.int32, (NUM_GRAPHS, NUM_GRAPHS), 1)
         ).astype(F32)
  offs = jnp.dot(cnts, tri, preferred_element_type=F32)
  off_node = lax.dot_general(Bh, offs, (((1,), (1,)), ((), ())),
                             preferred_element_type=F32)
  rowid = lax.broadcasted_iota(jnp.int32, (N, 1), 0).astype(F32)
  pos = rowid - off_node
  mask = pos < float(MAX_NODES)
  Pm = ((lax.broadcasted_iota(jnp.int32, (N, MAX_NODES), 1).astype(F32)
         == pos) & mask).astype(F32)
  out = lax.dot_general(Bh, Pm * preds, (((0,), (0,)), ((), ())),
                        preferred_element_type=F32)
  out_ref[...] = out


def _tc_call(body, out_shape):
  return pl.pallas_call(body, out_shape=out_shape)


def kernel(x, edge_index, counts, use_counts, batch, emb, cW1, cb1, cW2, cb2,
           conv_W1, conv_b1, conv_W2, conv_b2, conv_gamma, conv_beta,
           dW1, db1, dW2, db2):
  x2 = x.reshape(N, 1)
  batch2 = batch.reshape(N, 1)
  uc = jnp.asarray(use_counts, F32).reshape(1, 1)
  pad = jnp.arange(_NPAD, dtype=jnp.int32)
  srci = jnp.concatenate([edge_index[0], pad]).reshape(_NT, _CPT, _K)
  dsti = jnp.concatenate([edge_index[1], pad])
  W1a = conv_W1[0][:HID]
  W1b = conv_W1[0][HID:]

  p = _tc_call(_enc_body, jax.ShapeDtypeStruct((N, HID), F32))(
      x2, counts, uc, emb, cW1, cb1.reshape(1, -1), cW2, cb2.reshape(1, -1),
      W1a, W1b)

  h = None
  for i in range(NUM_LAYERS):
    agg0, agg1 = _segsum(p, srci, dsti)
    residual = i > 0
    last = i == NUM_LAYERS - 1
    body = functools.partial(_layer_body, residual=residual, last=last)
    if last:
      out_shape = jax.ShapeDtypeStruct((N, HID), F32)
    else:
      out_shape = (jax.ShapeDtypeStruct((N, HID), F32),
                   jax.ShapeDtypeStruct((N, HID), F32))
    args = []
    if residual:
      args.append(h)
    args += [p, agg0, agg1, conv_b1[i].reshape(1, -1), conv_W2[i],
             conv_b2[i].reshape(1, -1), conv_gamma[i].reshape(1, -1),
             conv_beta[i].reshape(1, -1)]
    if not last:
      args.append(conv_W1[i + 1])
      h, p = _tc_call(body, out_shape)(*args)
    else:
      h = _tc_call(body, out_shape)(*args)

  out = _tc_call(_final_body,
                 jax.ShapeDtypeStruct((NUM_GRAPHS, MAX_NODES), F32))(
      h, batch2, dW1, db1.reshape(1, -1), dW2, db2.reshape(1, -1))
  return out


# speedup vs baseline: 3.7949x; 1.0015x over previous
import functools

import jax
import jax.numpy as jnp
from jax import lax
from jax.experimental import pallas as pl
from jax.experimental.pallas import tpu as pltpu
from jax.experimental.pallas import tpu_sc as plsc

N = 10000
E = 320000
HID = 128
CNT = 16
NUM_LAYERS = 4
NUM_EMB = 121
MAX_NODES = 121
NUM_GRAPHS = 100

F32 = jnp.float32


_NC = 2
_NS = 16
_NT = _NC * _NS
_K = 64
_NSLOT = 3
_CPT = 159
_EPT = _CPT * _K
_EPAD = _NT * _EPT
_NPAD = _EPAD - E
_WCH = N // 80
_NITER = _CPT // _NSLOT


def _make_segsum():
  mesh = plsc.VectorSubcoreMesh(core_axis_name="c", subcore_axis_name="s")

  @functools.partial(
      pl.kernel,
      mesh=mesh,
      out_type=[
          jax.ShapeDtypeStruct((N, HID), F32),
          jax.ShapeDtypeStruct((N, HID), F32),
      ],
      scratch_types=(
          [pltpu.VMEM((_CPT, _K), jnp.int32)]
          + [pltpu.VMEM((_K,), jnp.int32)] * _NSLOT
          + [pltpu.VMEM((_K,), jnp.int32)] * _NSLOT
          + [pltpu.VMEM((_K, HID), F32)] * _NSLOT
          + [pltpu.VMEM((8, HID), F32)]
          + [pltpu.VMEM_SHARED((N, HID), F32)]
          + [pltpu.SemaphoreType.DMA] * (2 * _NSLOT)
          + [pltpu.SemaphoreType.DMA]
      ),
  )
  def segsum(p_hbm, packed_hbm, agg0_hbm, agg1_hbm, packedall, *rest):
    sbufs = rest[:_NSLOT]
    dbufs = rest[_NSLOT:2 * _NSLOT]
    rbufs = rest[2 * _NSLOT:3 * _NSLOT]
    zbuf = rest[3 * _NSLOT]
    acc = rest[3 * _NSLOT + 1]
    gsems = rest[3 * _NSLOT + 2:4 * _NSLOT + 2]
    ssems = rest[4 * _NSLOT + 2:5 * _NSLOT + 2]
    zsem = rest[5 * _NSLOT + 2]
    c = lax.axis_index("c")
    s = lax.axis_index("s")
    t = c * _NS + s

    pltpu.sync_copy(packed_hbm.at[t], packedall)

    def _gather(k, j):
      for u in range(_K // 16):
        v = packedall[k, pl.ds(u * 16, 16)]
        sbufs[j][pl.ds(u * 16, 16)] = jnp.bitwise_and(v, 0xFFFF)
        dbufs[j][pl.ds(u * 16, 16)] = lax.shift_right_logical(v, 16)
      pltpu.async_copy(p_hbm.at[sbufs[j]], rbufs[j], gsems[j])

    def _gwait(j):
      pltpu.make_async_copy(p_hbm.at[sbufs[j]], rbufs[j],
                            gsems[j]).wait()

    def _scat(j):
      return pltpu.async_copy(rbufs[j], acc.at[dbufs[j]], ssems[j],
                              add=True)

    for j in range(_NSLOT):
      _gather(j, j)

    def _zrow(r, _):
      def _zcol(j, _):
        zbuf[r, pl.ds(j * 16, 16)] = jnp.zeros((16,), F32)
        return 0
      lax.fori_loop(0, HID // 16, _zcol, 0)
      return 0
    lax.fori_loop(0, 8, _zrow, 0)
    _ZN = N // 8
    _ZIT = (_ZN + _NS - 1) // _NS

    def _zchunk(j, _):
      idx = s + j * _NS
      @pl.when(idx < _ZN)
      def _():
        pltpu.async_copy(zbuf, acc.at[pl.ds(idx * 8, 8)], zsem)
      return 0
    lax.fori_loop(0, _ZIT, _zchunk, 0)

    def _zdrain(j, _):
      idx = s + j * _NS
      @pl.when(idx < _ZN)
      def _():
        pltpu.make_async_copy(zbuf, acc.at[pl.ds(0, 8)], zsem).wait()
      return 0
    lax.fori_loop(0, _ZIT, _zdrain, 0)
    plsc.subcore_barrier()

    def _iter(i, _):
      k = _NSLOT * i
      handles = []
      for j in range(_NSLOT):
        _gwait(j)
        handles.append(_scat(j))
      for j in range(_NSLOT):
        handles[j].wait()
        @pl.when(i < _NITER - 1)
        def _(j=j):
          _gather(k + _NSLOT + j, j)
      return 0
    lax.fori_loop(0, _NITER, _iter, 0)
    plsc.subcore_barrier()

    _WN = N // 400
    _WIT = (_WN + _NS - 1) // _NS

    def _writeout(out_hbm):
      def _w(j, _):
        idx = s + j * _NS
        @pl.when(idx < _WN)
        def _():
          r0 = idx * 400
          pltpu.async_copy(acc.at[pl.ds(r0, 400)],
                           out_hbm.at[pl.ds(r0, 400)], zsem)
        return 0
      lax.fori_loop(0, _WIT, _w, 0)

      def _wd(j, _):
        idx = s + j * _NS
        @pl.when(idx < _WN)
        def _():
          pltpu.make_async_copy(acc.at[pl.ds(0, 400)],
                                out_hbm.at[pl.ds(0, 400)], zsem).wait()
        return 0
      lax.fori_loop(0, _WIT, _wd, 0)

    @pl.when(c == 0)
    def _():
      _writeout(agg0_hbm)

    @pl.when(c == 1)
    def _():
      _writeout(agg1_hbm)

  return segsum


_segsum_call = None


def _segsum(p, packed):
  global _segsum_call
  if _segsum_call is None:
    _segsum_call = _make_segsum()
  return _segsum_call(p, packed)


def _enc_body(x_ref, counts_ref, uc_ref, emb_ref, cW1_ref, cb1_ref,
              cW2_ref, cb2_ref, W1a_ref, W1b_ref, out_ref):
  xi = x_ref[...]
  onehot = (lax.broadcasted_iota(jnp.int32, (N, NUM_EMB), 1) == xi
            ).astype(F32)
  he = jnp.dot(onehot, emb_ref[...], preferred_element_type=F32)
  ch = jnp.maximum(
      jnp.dot(counts_ref[...], cW1_ref[...], preferred_element_type=F32)
      + cb1_ref[...], 0.0)
  ch = jnp.dot(ch, cW2_ref[...], preferred_element_type=F32) + cb2_ref[...]
  ch = ch * uc_ref[0, 0]
  out_ref[...] = (
      jnp.dot(he, W1a_ref[...], preferred_element_type=F32)
      + jnp.dot(ch, W1b_ref[...], preferred_element_type=F32))


def _layer_body(*refs, residual, last):
  if residual:
    h_ref, p_ref, a0_ref, a1_ref, b1_ref, W2_ref, b2_ref, g_ref, be_ref = \
        refs[:9]
    rest = refs[9:]
  else:
    p_ref, a0_ref, a1_ref, b1_ref, W2_ref, b2_ref, g_ref, be_ref = refs[:8]
    rest = refs[8:]
  if last:
    (out_ref,) = rest
  else:
    W1n_ref, out_ref, pout_ref = rest

  p = p_ref[...]
  corr = (lax.broadcasted_iota(jnp.int32, (N, 1), 0) < _NPAD).astype(F32)
  a1 = a1_ref[...] - corr * p
  z = jnp.maximum(p + a0_ref[...] + a1 + b1_ref[...], 0.0)
  z = jnp.dot(z, W2_ref[...], preferred_element_type=F32) + b2_ref[...]
  mu = jnp.mean(z, axis=0, keepdims=True)
  var = jnp.mean(jnp.square(z - mu), axis=0, keepdims=True)
  z = g_ref[...] * (z - mu) * lax.rsqrt(var + 1e-5) + be_ref[...]
  z = jnp.maximum(z, 0.0)
  if residual:
    z = z + h_ref[...]
  out_ref[...] = z
  if not last:
    pout_ref[...] = jnp.dot(z, W1n_ref[...], preferred_element_type=F32)


def _final_body(h_ref, batch_ref, dW1_ref, db1_ref, dW2_ref, db2_ref,
                out_ref):
  h = h_ref[...]
  z = jnp.maximum(
      jnp.dot(h, dW1_ref[...], preferred_element_type=F32) + db1_ref[...],
      0.0)
  z = jnp.dot(z, dW2_ref[...], preferred_element_type=F32) + db2_ref[...]
  preds = 1.0 / (1.0 + jnp.exp(-z))

  b = batch_ref[...]
  Bh = (lax.broadcasted_iota(jnp.int32, (N, NUM_GRAPHS), 1) == b
        ).astype(F32)
  cnts = jnp.sum(Bh, axis=0, keepdims=True)
  tri = (lax.broadcasted_iota(jnp.int32, (NUM_GRAPHS, NUM_GRAPHS), 0)
         < lax.broadcasted_iota(jnp.int32, (NUM_GRAPHS, NUM_GRAPHS), 1)
         ).astype(F32)
  offs = jnp.dot(cnts, tri, preferred_element_type=F32)
  off_node = lax.dot_general(Bh, offs, (((1,), (1,)), ((), ())),
                             preferred_element_type=F32)
  rowid = lax.broadcasted_iota(jnp.int32, (N, 1), 0).astype(F32)
  pos = rowid - off_node
  mask = pos < float(MAX_NODES)
  Pm = ((lax.broadcasted_iota(jnp.int32, (N, MAX_NODES), 1).astype(F32)
         == pos) & mask).astype(F32)
  out = lax.dot_general(Bh, Pm * preds, (((0,), (0,)), ((), ())),
                        preferred_element_type=F32)
  out_ref[...] = out


def _tc_call(body, out_shape):
  return pl.pallas_call(body, out_shape=out_shape)


def kernel(x, edge_index, counts, use_counts, batch, emb, cW1, cb1, cW2, cb2,
           conv_W1, conv_b1, conv_W2, conv_b2, conv_gamma, conv_beta,
           dW1, db1, dW2, db2):
  x2 = x.reshape(N, 1)
  batch2 = batch.reshape(N, 1)
  uc = jnp.asarray(use_counts, F32).reshape(1, 1)
  pad = jnp.arange(_NPAD, dtype=jnp.int32)
  packed = jnp.concatenate([
      edge_index[0] | (edge_index[1] << 16),
      pad | (pad << 16)]).reshape(_NT, _CPT, _K)
  W1a = conv_W1[0][:HID]
  W1b = conv_W1[0][HID:]

  p = _tc_call(_enc_body, jax.ShapeDtypeStruct((N, HID), F32))(
      x2, counts, uc, emb, cW1, cb1.reshape(1, -1), cW2, cb2.reshape(1, -1),
      W1a, W1b)

  h = None
  for i in range(NUM_LAYERS):
    agg0, agg1 = _segsum(p, packed)
    residual = i > 0
    last = i == NUM_LAYERS - 1
    body = functools.partial(_layer_body, residual=residual, last=last)
    if last:
      out_shape = jax.ShapeDtypeStruct((N, HID), F32)
    else:
      out_shape = (jax.ShapeDtypeStruct((N, HID), F32),
                   jax.ShapeDtypeStruct((N, HID), F32))
    args = []
    if residual:
      args.append(h)
    args += [p, agg0, agg1, conv_b1[i].reshape(1, -1), conv_W2[i],
             conv_b2[i].reshape(1, -1), conv_gamma[i].reshape(1, -1),
             conv_beta[i].reshape(1, -1)]
    if not last:
      args.append(conv_W1[i + 1])
      h, p = _tc_call(body, out_shape)(*args)
    else:
      h = _tc_call(body, out_shape)(*args)

  out = _tc_call(_final_body,
                 jax.ShapeDtypeStruct((NUM_GRAPHS, MAX_NODES), F32))(
      h, batch2, dW1, db1.reshape(1, -1), dW2, db2.reshape(1, -1))
  return out


# speedup vs baseline: 3.8382x; 1.0114x over previous
import functools

import jax
import jax.numpy as jnp
from jax import lax
from jax.experimental import pallas as pl
from jax.experimental.pallas import tpu as pltpu
from jax.experimental.pallas import tpu_sc as plsc

N = 10000
E = 320000
HID = 128
CNT = 16
NUM_LAYERS = 4
NUM_EMB = 121
MAX_NODES = 121
NUM_GRAPHS = 100

F32 = jnp.float32


_NC = 2
_NS = 16
_NT = _NC * _NS
_K = 64
_NSLOT = 3
_CPT = 159
_EPT = _CPT * _K
_EPAD = _NT * _EPT
_NPAD = _EPAD - E
_WCH = N // 80
_NITER = _CPT // _NSLOT


def _make_segsum():
  mesh = plsc.VectorSubcoreMesh(core_axis_name="c", subcore_axis_name="s")

  @functools.partial(
      pl.kernel,
      mesh=mesh,
      out_type=[
          jax.ShapeDtypeStruct((N, HID), F32),
          jax.ShapeDtypeStruct((N, HID), F32),
      ],
      scratch_types=(
          [pltpu.VMEM((_CPT, _K), jnp.int32)]
          + [pltpu.VMEM((_K,), jnp.int32)] * _NSLOT
          + [pltpu.VMEM((_K,), jnp.int32)] * _NSLOT
          + [pltpu.VMEM((_K, HID), F32)] * _NSLOT
          + [pltpu.VMEM((8, HID), F32)]
          + [pltpu.VMEM_SHARED((N, HID), F32)]
          + [pltpu.SemaphoreType.DMA] * (2 * _NSLOT)
          + [pltpu.SemaphoreType.DMA]
      ),
  )
  def segsum(p_hbm, packed_hbm, agg0_hbm, agg1_hbm, packedall, *rest):
    sbufs = rest[:_NSLOT]
    dbufs = rest[_NSLOT:2 * _NSLOT]
    rbufs = rest[2 * _NSLOT:3 * _NSLOT]
    zbuf = rest[3 * _NSLOT]
    acc = rest[3 * _NSLOT + 1]
    gsems = rest[3 * _NSLOT + 2:4 * _NSLOT + 2]
    ssems = rest[4 * _NSLOT + 2:5 * _NSLOT + 2]
    zsem = rest[5 * _NSLOT + 2]
    c = lax.axis_index("c")
    s = lax.axis_index("s")
    t = c * _NS + s

    pltpu.sync_copy(packed_hbm.at[t], packedall)

    def _gather(k, j):
      for u in range(_K // 16):
        v = packedall[k, pl.ds(u * 16, 16)]
        sbufs[j][pl.ds(u * 16, 16)] = jnp.bitwise_and(v, 0xFFFF)
        dbufs[j][pl.ds(u * 16, 16)] = lax.shift_right_logical(v, 16)
      pltpu.async_copy(p_hbm.at[sbufs[j]], rbufs[j], gsems[j])

    def _gwait(j):
      pltpu.make_async_copy(p_hbm.at[sbufs[j]], rbufs[j],
                            gsems[j]).wait()

    def _scat(j):
      return pltpu.async_copy(rbufs[j], acc.at[dbufs[j]], ssems[j],
                              add=True)

    for j in range(_NSLOT):
      _gather(j, j)

    def _zrow(r, _):
      def _zcol(j, _):
        zbuf[r, pl.ds(j * 16, 16)] = jnp.zeros((16,), F32)
        return 0
      lax.fori_loop(0, HID // 16, _zcol, 0)
      return 0
    lax.fori_loop(0, 8, _zrow, 0)
    _ZN = N // 8
    _ZIT = (_ZN + _NS - 1) // _NS

    def _zchunk(j, _):
      idx = s + j * _NS
      @pl.when(idx < _ZN)
      def _():
        pltpu.async_copy(zbuf, acc.at[pl.ds(idx * 8, 8)], zsem)
      return 0
    lax.fori_loop(0, _ZIT, _zchunk, 0)

    def _zdrain(j, _):
      idx = s + j * _NS
      @pl.when(idx < _ZN)
      def _():
        pltpu.make_async_copy(zbuf, acc.at[pl.ds(0, 8)], zsem).wait()
      return 0
    lax.fori_loop(0, _ZIT, _zdrain, 0)
    plsc.subcore_barrier()

    def _iter(i, _):
      k = _NSLOT * i
      handles = []
      for j in range(_NSLOT):
        _gwait(j)
        handles.append(_scat(j))
      for j in range(_NSLOT):
        handles[j].wait()
        @pl.when(i < _NITER - 1)
        def _(j=j):
          _gather(k + _NSLOT + j, j)
      return 0
    lax.fori_loop(0, _NITER, _iter, 0)
    plsc.subcore_barrier()

    _WN = N // 400
    _WIT = (_WN + _NS - 1) // _NS

    def _writeout(out_hbm):
      def _w(j, _):
        idx = s + j * _NS
        @pl.when(idx < _WN)
        def _():
          r0 = idx * 400
          pltpu.async_copy(acc.at[pl.ds(r0, 400)],
                           out_hbm.at[pl.ds(r0, 400)], zsem)
        return 0
      lax.fori_loop(0, _WIT, _w, 0)

      def _wd(j, _):
        idx = s + j * _NS
        @pl.when(idx < _WN)
        def _():
          pltpu.make_async_copy(acc.at[pl.ds(0, 400)],
                                out_hbm.at[pl.ds(0, 400)], zsem).wait()
        return 0
      lax.fori_loop(0, _WIT, _wd, 0)

    @pl.when(c == 0)
    def _():
      _writeout(agg0_hbm)

    @pl.when(c == 1)
    def _():
      _writeout(agg1_hbm)

  return segsum


_segsum_call = None


def _segsum(p, packed):
  global _segsum_call
  if _segsum_call is None:
    _segsum_call = _make_segsum()
  return _segsum_call(p, packed)


def _enc_body(x_ref, counts_ref, uc_ref, emb_ref, cW1_ref, cb1_ref,
              cW2_ref, cb2_ref, W1a_ref, W1b_ref, out_ref):
  xi = x_ref[...]
  onehot = (lax.broadcasted_iota(jnp.int32, (N, NUM_EMB), 1) == xi
            ).astype(F32)
  he = jnp.dot(onehot, emb_ref[...], preferred_element_type=F32)
  ch = jnp.maximum(
      jnp.dot(counts_ref[...], cW1_ref[...], preferred_element_type=F32)
      + cb1_ref[...], 0.0)
  ch = jnp.dot(ch, cW2_ref[...], preferred_element_type=F32) + cb2_ref[...]
  ch = ch * uc_ref[0, 0]
  out_ref[...] = (
      jnp.dot(he, W1a_ref[...], preferred_element_type=F32)
      + jnp.dot(ch, W1b_ref[...], preferred_element_type=F32))


def _layer_body(*refs, residual, last):
  if residual:
    h_ref, p_ref, a0_ref, a1_ref, b1_ref, W2_ref, b2_ref, g_ref, be_ref = \
        refs[:9]
    rest = refs[9:]
  else:
    p_ref, a0_ref, a1_ref, b1_ref, W2_ref, b2_ref, g_ref, be_ref = refs[:8]
    rest = refs[8:]
  if last:
    (out_ref,) = rest
  else:
    W1n_ref, out_ref, pout_ref = rest

  p = p_ref[...]
  corr = (lax.broadcasted_iota(jnp.int32, (N, 1), 0) < _NPAD).astype(F32)
  a1 = a1_ref[...] - corr * p
  z = jnp.maximum(p + a0_ref[...] + a1 + b1_ref[...], 0.0)
  z = jnp.dot(z, W2_ref[...], preferred_element_type=F32) + b2_ref[...]
  mu = jnp.mean(z, axis=0, keepdims=True)
  var = jnp.mean(jnp.square(z - mu), axis=0, keepdims=True)
  z = g_ref[...] * (z - mu) * lax.rsqrt(var + 1e-5) + be_ref[...]
  z = jnp.maximum(z, 0.0)
  if residual:
    z = z + h_ref[...]
  out_ref[...] = z
  if not last:
    pout_ref[...] = jnp.dot(z, W1n_ref[...], preferred_element_type=F32)


def _last_body(h_ref, p_ref, a0_ref, a1_ref, b1_ref, W2_ref, b2_ref,
               g_ref, be_ref, batch_ref, dW1_ref, db1_ref, dW2_ref,
               db2_ref, out_ref):
  p = p_ref[...]
  corr = (lax.broadcasted_iota(jnp.int32, (N, 1), 0) < _NPAD).astype(F32)
  a1 = a1_ref[...] - corr * p
  z = jnp.maximum(p + a0_ref[...] + a1 + b1_ref[...], 0.0)
  z = jnp.dot(z, W2_ref[...], preferred_element_type=F32) + b2_ref[...]
  mu = jnp.mean(z, axis=0, keepdims=True)
  var = jnp.mean(jnp.square(z - mu), axis=0, keepdims=True)
  z = g_ref[...] * (z - mu) * lax.rsqrt(var + 1e-5) + be_ref[...]
  h = jnp.maximum(z, 0.0) + h_ref[...]
  z = jnp.maximum(
      jnp.dot(h, dW1_ref[...], preferred_element_type=F32) + db1_ref[...],
      0.0)
  z = jnp.dot(z, dW2_ref[...], preferred_element_type=F32) + db2_ref[...]
  preds = 1.0 / (1.0 + jnp.exp(-z))

  b = batch_ref[...]
  Bh = (lax.broadcasted_iota(jnp.int32, (N, NUM_GRAPHS), 1) == b
        ).astype(F32)
  cnts = jnp.sum(Bh, axis=0, keepdims=True)
  tri = (lax.broadcasted_iota(jnp.int32, (NUM_GRAPHS, NUM_GRAPHS), 0)
         < lax.broadcasted_iota(jnp.int32, (NUM_GRAPHS, NUM_GRAPHS), 1)
         ).astype(F32)
  offs = jnp.dot(cnts, tri, preferred_element_type=F32)
  off_node = lax.dot_general(Bh, offs, (((1,), (1,)), ((), ())),
                             preferred_element_type=F32)
  rowid = lax.broadcasted_iota(jnp.int32, (N, 1), 0).astype(F32)
  pos = rowid - off_node
  mask = pos < float(MAX_NODES)
  Pm = ((lax.broadcasted_iota(jnp.int32, (N, MAX_NODES), 1).astype(F32)
         == pos) & mask).astype(F32)
  out = lax.dot_general(Bh, Pm * preds, (((0,), (0,)), ((), ())),
                        preferred_element_type=F32)
  out_ref[...] = out


def _tc_call(body, out_shape):
  return pl.pallas_call(body, out_shape=out_shape)


def kernel(x, edge_index, counts, use_counts, batch, emb, cW1, cb1, cW2, cb2,
           conv_W1, conv_b1, conv_W2, conv_b2, conv_gamma, conv_beta,
           dW1, db1, dW2, db2):
  x2 = x.reshape(N, 1)
  batch2 = batch.reshape(N, 1)
  uc = jnp.asarray(use_counts, F32).reshape(1, 1)
  pad = jnp.arange(_NPAD, dtype=jnp.int32)
  packed = jnp.concatenate([
      edge_index[0] | (edge_index[1] << 16),
      pad | (pad << 16)]).reshape(_NT, _CPT, _K)
  W1a = conv_W1[0][:HID]
  W1b = conv_W1[0][HID:]

  p = _tc_call(_enc_body, jax.ShapeDtypeStruct((N, HID), F32))(
      x2, counts, uc, emb, cW1, cb1.reshape(1, -1), cW2, cb2.reshape(1, -1),
      W1a, W1b)

  h = None
  for i in range(NUM_LAYERS - 1):
    agg0, agg1 = _segsum(p, packed)
    residual = i > 0
    body = functools.partial(_layer_body, residual=residual, last=False)
    out_shape = (jax.ShapeDtypeStruct((N, HID), F32),
                 jax.ShapeDtypeStruct((N, HID), F32))
    args = []
    if residual:
      args.append(h)
    args += [p, agg0, agg1, conv_b1[i].reshape(1, -1), conv_W2[i],
             conv_b2[i].reshape(1, -1), conv_gamma[i].reshape(1, -1),
             conv_beta[i].reshape(1, -1), conv_W1[i + 1]]
    h, p = _tc_call(body, out_shape)(*args)

  i = NUM_LAYERS - 1
  agg0, agg1 = _segsum(p, packed)
  out = _tc_call(_last_body,
                 jax.ShapeDtypeStruct((NUM_GRAPHS, MAX_NODES), F32))(
      h, p, agg0, agg1, conv_b1[i].reshape(1, -1), conv_W2[i],
      conv_b2[i].reshape(1, -1), conv_gamma[i].reshape(1, -1),
      conv_beta[i].reshape(1, -1), batch2,
      dW1, db1.reshape(1, -1), dW2, db2.reshape(1, -1))
  return out


# speedup vs baseline: 3.9101x; 1.0187x over previous
import functools

import jax
import jax.numpy as jnp
from jax import lax
from jax.experimental import pallas as pl
from jax.experimental.pallas import tpu as pltpu
from jax.experimental.pallas import tpu_sc as plsc

N = 10000
E = 320000
HID = 128
CNT = 16
NUM_LAYERS = 4
NUM_EMB = 121
MAX_NODES = 121
NUM_GRAPHS = 100

F32 = jnp.float32


_NC = 2
_NS = 16
_NT = _NC * _NS
_K = 64
_NSLOT = 3
_CPT = 159
_EPT = _CPT * _K
_EPAD = _NT * _EPT
_NPAD = _EPAD - E
_WCH = N // 80
_NITER = _CPT // _NSLOT


def _make_segsum():
  mesh = plsc.VectorSubcoreMesh(core_axis_name="c", subcore_axis_name="s")

  @functools.partial(
      pl.kernel,
      mesh=mesh,
      out_type=[
          jax.ShapeDtypeStruct((N, HID), F32),
          jax.ShapeDtypeStruct((N, HID), F32),
      ],
      scratch_types=(
          [pltpu.VMEM((_EPT,), jnp.int32)]
          + [pltpu.VMEM((_K,), jnp.int32)] * _NSLOT
          + [pltpu.VMEM((_K,), jnp.int32)] * _NSLOT
          + [pltpu.VMEM((_K, HID), F32)] * _NSLOT
          + [pltpu.VMEM((8, HID), F32)]
          + [pltpu.VMEM_SHARED((N, HID), F32)]
          + [pltpu.SemaphoreType.DMA] * (2 * _NSLOT)
          + [pltpu.SemaphoreType.DMA]
      ),
  )
  def segsum(p_hbm, packed_hbm, agg0_hbm, agg1_hbm, packedall, *rest):
    sbufs = rest[:_NSLOT]
    dbufs = rest[_NSLOT:2 * _NSLOT]
    rbufs = rest[2 * _NSLOT:3 * _NSLOT]
    zbuf = rest[3 * _NSLOT]
    acc = rest[3 * _NSLOT + 1]
    gsems = rest[3 * _NSLOT + 2:4 * _NSLOT + 2]
    ssems = rest[4 * _NSLOT + 2:5 * _NSLOT + 2]
    zsem = rest[5 * _NSLOT + 2]
    c = lax.axis_index("c")
    s = lax.axis_index("s")
    t = c * _NS + s

    pltpu.sync_copy(packed_hbm.at[pl.ds(t * _EPT, _EPT)], packedall)

    def _gather(k, j):
      for u in range(_K // 16):
        v = packedall[pl.ds(k * _K + u * 16, 16)]
        sbufs[j][pl.ds(u * 16, 16)] = jnp.bitwise_and(v, 0xFFFF)
        dbufs[j][pl.ds(u * 16, 16)] = lax.shift_right_logical(v, 16)
      pltpu.async_copy(p_hbm.at[sbufs[j]], rbufs[j], gsems[j])

    def _gwait(j):
      pltpu.make_async_copy(p_hbm.at[sbufs[j]], rbufs[j],
                            gsems[j]).wait()

    def _scat(j):
      return pltpu.async_copy(rbufs[j], acc.at[dbufs[j]], ssems[j],
                              add=True)

    for j in range(_NSLOT):
      _gather(j, j)

    def _zrow(r, _):
      def _zcol(j, _):
        zbuf[r, pl.ds(j * 16, 16)] = jnp.zeros((16,), F32)
        return 0
      lax.fori_loop(0, HID // 16, _zcol, 0)
      return 0
    lax.fori_loop(0, 8, _zrow, 0)
    _ZN = N // 8
    _ZIT = (_ZN + _NS - 1) // _NS

    def _zchunk(j, _):
      idx = s + j * _NS
      @pl.when(idx < _ZN)
      def _():
        pltpu.async_copy(zbuf, acc.at[pl.ds(idx * 8, 8)], zsem)
      return 0
    lax.fori_loop(0, _ZIT, _zchunk, 0)

    def _zdrain(j, _):
      idx = s + j * _NS
      @pl.when(idx < _ZN)
      def _():
        pltpu.make_async_copy(zbuf, acc.at[pl.ds(0, 8)], zsem).wait()
      return 0
    lax.fori_loop(0, _ZIT, _zdrain, 0)
    plsc.subcore_barrier()

    def _iter(i, _):
      k = _NSLOT * i
      handles = []
      for j in range(_NSLOT):
        _gwait(j)
        handles.append(_scat(j))
      for j in range(_NSLOT):
        handles[j].wait()
        @pl.when(i < _NITER - 1)
        def _(j=j):
          _gather(k + _NSLOT + j, j)
      return 0
    lax.fori_loop(0, _NITER, _iter, 0)
    plsc.subcore_barrier()

    _WN = N // 400
    _WIT = (_WN + _NS - 1) // _NS

    def _writeout(out_hbm):
      def _w(j, _):
        idx = s + j * _NS
        @pl.when(idx < _WN)
        def _():
          r0 = idx * 400
          pltpu.async_copy(acc.at[pl.ds(r0, 400)],
                           out_hbm.at[pl.ds(r0, 400)], zsem)
        return 0
      lax.fori_loop(0, _WIT, _w, 0)

      def _wd(j, _):
        idx = s + j * _NS
        @pl.when(idx < _WN)
        def _():
          pltpu.make_async_copy(acc.at[pl.ds(0, 400)],
                                out_hbm.at[pl.ds(0, 400)], zsem).wait()
        return 0
      lax.fori_loop(0, _WIT, _wd, 0)

    @pl.when(c == 0)
    def _():
      _writeout(agg0_hbm)

    @pl.when(c == 1)
    def _():
      _writeout(agg1_hbm)

  return segsum


_segsum_call = None


def _segsum(p, packed):
  global _segsum_call
  if _segsum_call is None:
    _segsum_call = _make_segsum()
  return _segsum_call(p, packed)


_EROW = E // 128
_PROW = _EPAD // 128


def _enc_body(x_ref, counts_ref, uc_ref, emb_ref, cW1_ref, cb1_ref,
              cW2_ref, cb2_ref, W1_ref, src_ref, dst_ref,
              out_ref, pk_ref):
  xi = x_ref[...]
  onehot = (lax.broadcasted_iota(jnp.int32, (NUM_EMB, N), 0) == xi
            ).astype(F32)
  he = lax.dot_general(onehot, emb_ref[...], (((0,), (0,)), ((), ())),
                       preferred_element_type=F32)
  ch = jnp.maximum(
      jnp.dot(counts_ref[...], cW1_ref[...], preferred_element_type=F32)
      + cb1_ref[...], 0.0)
  ch = jnp.dot(ch, cW2_ref[...], preferred_element_type=F32) + cb2_ref[...]
  ch = ch * uc_ref[0, 0]
  W1 = W1_ref[...]
  out_ref[...] = (
      jnp.dot(he, W1[:HID], preferred_element_type=F32)
      + jnp.dot(ch, W1[HID:], preferred_element_type=F32))
  pk_ref[pl.ds(0, _EROW), :] = src_ref[...] | (dst_ref[...] << 16)
  padidx = (lax.broadcasted_iota(jnp.int32, (_PROW - _EROW, 128), 0) * 128
            + lax.broadcasted_iota(jnp.int32, (_PROW - _EROW, 128), 1))
  pk_ref[pl.ds(_EROW, _PROW - _EROW), :] = padidx | (padidx << 16)


def _layer_body(*refs, residual, last):
  if residual:
    h_ref, p_ref, a0_ref, a1_ref, b1_ref, W2_ref, b2_ref, g_ref, be_ref = \
        refs[:9]
    rest = refs[9:]
  else:
    p_ref, a0_ref, a1_ref, b1_ref, W2_ref, b2_ref, g_ref, be_ref = refs[:8]
    rest = refs[8:]
  if last:
    (out_ref,) = rest
  else:
    W1n_ref, out_ref, pout_ref = rest

  p = p_ref[...]
  corr = (lax.broadcasted_iota(jnp.int32, (N, 1), 0) < _NPAD).astype(F32)
  a1 = a1_ref[...] - corr * p
  z = jnp.maximum(p + a0_ref[...] + a1 + b1_ref[...], 0.0)
  z = jnp.dot(z, W2_ref[...], preferred_element_type=F32) + b2_ref[...]
  mu = jnp.mean(z, axis=0, keepdims=True)
  var = jnp.mean(jnp.square(z - mu), axis=0, keepdims=True)
  z = g_ref[...] * (z - mu) * lax.rsqrt(var + 1e-5) + be_ref[...]
  z = jnp.maximum(z, 0.0)
  if residual:
    z = z + h_ref[...]
  out_ref[...] = z
  if not last:
    pout_ref[...] = jnp.dot(z, W1n_ref[...], preferred_element_type=F32)


def _last_body(h_ref, p_ref, a0_ref, a1_ref, b1_ref, W2_ref, b2_ref,
               g_ref, be_ref, batch_ref, dW1_ref, db1_ref, dW2_ref,
               db2_ref, out_ref):
  p = p_ref[...]
  corr = (lax.broadcasted_iota(jnp.int32, (N, 1), 0) < _NPAD).astype(F32)
  a1 = a1_ref[...] - corr * p
  z = jnp.maximum(p + a0_ref[...] + a1 + b1_ref[...], 0.0)
  z = jnp.dot(z, W2_ref[...], preferred_element_type=F32) + b2_ref[...]
  mu = jnp.mean(z, axis=0, keepdims=True)
  var = jnp.mean(jnp.square(z - mu), axis=0, keepdims=True)
  z = g_ref[...] * (z - mu) * lax.rsqrt(var + 1e-5) + be_ref[...]
  h = jnp.maximum(z, 0.0) + h_ref[...]
  z = jnp.maximum(
      jnp.dot(h, dW1_ref[...], preferred_element_type=F32) + db1_ref[...],
      0.0)
  z = jnp.dot(z, dW2_ref[...], preferred_element_type=F32) + db2_ref[...]
  preds = 1.0 / (1.0 + jnp.exp(-z))

  b = batch_ref[...]
  BhT = (lax.broadcasted_iota(jnp.int32, (NUM_GRAPHS, N), 0) == b
         ).astype(F32)
  cnts = jnp.sum(BhT, axis=1, keepdims=True)
  tri = (lax.broadcasted_iota(jnp.int32, (NUM_GRAPHS, NUM_GRAPHS), 1)
         < lax.broadcasted_iota(jnp.int32, (NUM_GRAPHS, NUM_GRAPHS), 0)
         ).astype(F32)
  offs = jnp.dot(tri, cnts, preferred_element_type=F32)
  off_node = lax.dot_general(BhT, offs, (((0,), (0,)), ((), ())),
                             preferred_element_type=F32)
  rowid = lax.broadcasted_iota(jnp.int32, (N, 1), 0).astype(F32)
  pos = rowid - off_node
  mask = pos < float(MAX_NODES)
  Pm = ((lax.broadcasted_iota(jnp.int32, (N, MAX_NODES), 1).astype(F32)
         == pos) & mask).astype(F32)
  out = jnp.dot(BhT, Pm * preds, preferred_element_type=F32)
  out_ref[...] = out


def _tc_call(body, out_shape):
  return pl.pallas_call(body, out_shape=out_shape)


def kernel(x, edge_index, counts, use_counts, batch, emb, cW1, cb1, cW2, cb2,
           conv_W1, conv_b1, conv_W2, conv_b2, conv_gamma, conv_beta,
           dW1, db1, dW2, db2):
  x2 = x.reshape(1, N)
  batch2 = batch.reshape(1, N)
  uc = jnp.asarray(use_counts, F32).reshape(1, 1)
  src2 = edge_index[0].reshape(_EROW, 128)
  dst2 = edge_index[1].reshape(_EROW, 128)

  p, packed2 = _tc_call(
      _enc_body, (jax.ShapeDtypeStruct((N, HID), F32),
                  jax.ShapeDtypeStruct((_PROW, 128), jnp.int32)))(
      x2, counts, uc, emb, cW1, cb1.reshape(1, -1), cW2, cb2.reshape(1, -1),
      conv_W1[0], src2, dst2)
  packed = packed2.reshape(_EPAD)

  h = None
  for i in range(NUM_LAYERS - 1):
    agg0, agg1 = _segsum(p, packed)
    residual = i > 0
    body = functools.partial(_layer_body, residual=residual, last=False)
    out_shape = (jax.ShapeDtypeStruct((N, HID), F32),
                 jax.ShapeDtypeStruct((N, HID), F32))
    args = []
    if residual:
      args.append(h)
    args += [p, agg0, agg1, conv_b1[i].reshape(1, -1), conv_W2[i],
             conv_b2[i].reshape(1, -1), conv_gamma[i].reshape(1, -1),
             conv_beta[i].reshape(1, -1), conv_W1[i + 1]]
    h, p = _tc_call(body, out_shape)(*args)

  i = NUM_LAYERS - 1
  agg0, agg1 = _segsum(p, packed)
  out = _tc_call(_last_body,
                 jax.ShapeDtypeStruct((NUM_GRAPHS, MAX_NODES), F32))(
      h, p, agg0, agg1, conv_b1[i].reshape(1, -1), conv_W2[i],
      conv_b2[i].reshape(1, -1), conv_gamma[i].reshape(1, -1),
      conv_beta[i].reshape(1, -1), batch2,
      dW1, db1.reshape(1, -1), dW2, db2.reshape(1, -1))
  return out
